# baseline (speedup 1.0000x reference)
import functools

import jax
import jax.numpy as jnp
from jax import lax
from jax.experimental import pallas as pl
from jax.experimental.pallas import tpu as pltpu
from jax.experimental.pallas import tpu_sc as plsc

_B, _F, _C, _N = 128, 512, 3, 66
_TOT_IN = _B * _F * _C * _N
_TOT_OUT = _TOT_IN // 2
_NW = 32
_IN_W = _TOT_IN // _NW
_OUT_W = _TOT_OUT // _NW
_NCH = 44
_CH_OUT = _OUT_W // _NCH
_CH_IN = 2 * _CH_OUT
_VREGS = _CH_OUT // 16
_UNROLL = 8


def _pair_sum_body(x_hbm, o_hbm, in0, in1, ot0, ot1, si0, si1, so0, so1):
    wid = lax.axis_index("s") * 2 + lax.axis_index("c")
    in_base = wid * _IN_W
    out_base = wid * _OUT_W

    l16 = lax.iota(jnp.int32, 16)
    hi = lax.shift_right_logical(l16, 3)
    idx_a = l16 + l16 + hi
    idx_b = idx_a + 1 - hi - hi

    def in_copy(ci, buf, sem):
        return pltpu.make_async_copy(
            x_hbm.at[pl.ds(in_base + ci * _CH_IN, _CH_IN)], buf, sem)

    def out_copy(ci, buf, sem):
        return pltpu.make_async_copy(
            buf, o_hbm.at[pl.ds(out_base + ci * _CH_OUT, _CH_OUT)], sem)

    def compute_chunk(in_b, out_b):
        def body(jb, carry):
            j0 = jb * _UNROLL
            for u in range(_UNROLL):
                j = j0 + u
                base = j * 32
                va = plsc.load_gather(in_b, [idx_a + base])
                vb = plsc.load_gather(in_b, [idx_b + base])
                out_b[pl.ds(j * 16, 16)] = va + vb
            return carry
        lax.fori_loop(0, _VREGS // _UNROLL, body, 0)

    bufs = ((in0, ot0, si0, so0), (in1, ot1, si1, so1))
    in_copy(0, in0, si0).start()
    in_copy(1, in1, si1).start()

    def pair(p, carry):
        for b in range(2):
            inb, otb, sib, sob = bufs[b]
            ci = p * 2 + b
            in_copy(ci, inb, sib).wait()

            @pl.when(p >= 1)
            def _wait_out():
                out_copy(ci - 2, otb, sob).wait()

            compute_chunk(inb, otb)
            out_copy(ci, otb, sob).start()

            @pl.when(ci + 2 < _NCH)
            def _next_in():
                in_copy(ci + 2, inb, sib).start()
        return carry

    lax.fori_loop(0, _NCH // 2, pair, 0)
    out_copy(_NCH - 2, ot0, so0).wait()
    out_copy(_NCH - 1, ot1, so1).wait()


_pair_sum = pl.kernel(
    _pair_sum_body,
    out_type=jax.ShapeDtypeStruct((_TOT_OUT,), jnp.float32),
    mesh=plsc.VectorSubcoreMesh(core_axis_name="c", subcore_axis_name="s"),
    compiler_params=pltpu.CompilerParams(needs_layout_passes=False),
    scratch_types=[
        pltpu.VMEM((_CH_IN,), jnp.float32),
        pltpu.VMEM((_CH_IN,), jnp.float32),
        pltpu.VMEM((_CH_OUT,), jnp.float32),
        pltpu.VMEM((_CH_OUT,), jnp.float32),
        pltpu.SemaphoreType.DMA,
        pltpu.SemaphoreType.DMA,
        pltpu.SemaphoreType.DMA,
        pltpu.SemaphoreType.DMA,
    ],
)


def kernel(x):
    out_flat = _pair_sum(x.reshape(-1))
    return out_flat.reshape(_B, _F, _C, _N // 2)


# speedup vs baseline: 7.0487x; 7.0487x over previous
import jax
import jax.numpy as jnp
from jax import lax
from jax.experimental import pallas as pl
from jax.experimental.pallas import tpu as pltpu
from jax.experimental.pallas import tpu_sc as plsc

_B, _F, _C, _N = 128, 512, 3, 66
_G = (_C * _N) // 2
_RC = 8
_NCHUNK = _B // _RC
_UNITS = _G * _NCHUNK
_NW = 32
_K = -(-_UNITS // _NW)


def _pair_slab_body(x_hbm, o_hbm, in0, in1, ot0, ot1, si0, si1, so0, so1):
    wid = lax.axis_index("s") * 2 + lax.axis_index("c")

    def unit_coords(k):
        u = wid + k * _NW
        g = lax.shift_right_logical(u, 4)
        r0 = lax.bitwise_and(u, 15) * _RC
        return u, g, r0

    def in_copy(k, buf, sem):
        _, g, r0 = unit_coords(k)
        return pltpu.make_async_copy(
            x_hbm.at[g, :, pl.ds(r0, _RC), :], buf, sem)

    def out_copy(k, buf, sem):
        _, g, r0 = unit_coords(k)
        return pltpu.make_async_copy(
            buf, o_hbm.at[g, pl.ds(r0, _RC), :], sem)

    def compute(in_b, out_b):
        def row(r, carry):
            for c in range(_F // 16):
                sl = pl.ds(c * 16, 16)
                out_b[r, sl] = in_b[0, r, sl] + in_b[1, r, sl]
            return carry
        lax.fori_loop(0, _RC, row, 0)

    bufs = ((in0, ot0, si0, so0), (in1, ot1, si1, so1))
    in_copy(0, in0, si0).start()
    in_copy(1, in1, si1).start()

    def pair(p, carry):
        for b in range(2):
            inb, otb, sib, sob = bufs[b]
            k = p * 2 + b
            u = wid + k * _NW
            valid = u < _UNITS

            @pl.when(valid)
            def _go():
                in_copy(k, inb, sib).wait()

                @pl.when(p >= 1)
                def _wait_out():
                    out_copy(k - 2, otb, sob).wait()

                compute(inb, otb)
                out_copy(k, otb, sob).start()

            @pl.when(u + 2 * _NW < _UNITS)
            def _next_in():
                in_copy(k + 2, inb, sib).start()
        return carry

    lax.fori_loop(0, _K // 2, pair, 0)
    out_copy(_K - 2, ot0, so0).wait()

    @pl.when(wid + (_K - 1) * _NW < _UNITS)
    def _drain_last():
        out_copy(_K - 1, ot1, so1).wait()


_pair_slab = pl.kernel(
    _pair_slab_body,
    out_type=jax.ShapeDtypeStruct((_G, _B, _F), jnp.float32),
    mesh=plsc.VectorSubcoreMesh(core_axis_name="c", subcore_axis_name="s"),
    compiler_params=pltpu.CompilerParams(needs_layout_passes=False),
    scratch_types=[
        pltpu.VMEM((2, _RC, _F), jnp.float32),
        pltpu.VMEM((2, _RC, _F), jnp.float32),
        pltpu.VMEM((_RC, _F), jnp.float32),
        pltpu.VMEM((_RC, _F), jnp.float32),
        pltpu.SemaphoreType.DMA,
        pltpu.SemaphoreType.DMA,
        pltpu.SemaphoreType.DMA,
        pltpu.SemaphoreType.DMA,
    ],
)


def kernel(x):
    xt = x.transpose(2, 3, 0, 1).reshape(_G, 2, _B, _F)
    out = _pair_slab(xt)
    return out.reshape(_C, _N // 2, _B, _F).transpose(2, 3, 0, 1)


# speedup vs baseline: 8.5058x; 1.2067x over previous
import jax
import jax.numpy as jnp
from jax import lax
from jax.experimental import pallas as pl
from jax.experimental.pallas import tpu as pltpu
from jax.experimental.pallas import tpu_sc as plsc

_B, _F, _C, _N = 128, 512, 3, 66
_G = (_C * _N) // 2
_RC = 8
_NCHUNK = _B // _RC
_UNITS = _G * _NCHUNK
_NW = 32
_NBI = 4
_NBO = 2
_K = 52


def _pair_slab_body(x_hbm, o_hbm, in0, in1, in2, in3, ot0, ot1,
                    si0, si1, si2, si3, so0, so1):
    wid = lax.axis_index("s") * 2 + lax.axis_index("c")
    ins = ((in0, si0), (in1, si1), (in2, si2), (in3, si3))
    outs = ((ot0, so0), (ot1, so1))

    def unit_coords(k):
        u = wid + k * _NW
        g = lax.shift_right_logical(u, 4)
        r0 = lax.bitwise_and(u, 15) * _RC
        return u, g, r0

    def in_copy(k, slot):
        _, g, r0 = unit_coords(k)
        buf, sem = ins[slot]
        return pltpu.make_async_copy(
            x_hbm.at[g, :, pl.ds(r0, _RC), :], buf, sem)

    def out_copy(k, slot):
        _, g, r0 = unit_coords(k)
        buf, sem = outs[slot]
        return pltpu.make_async_copy(
            buf, o_hbm.at[g, pl.ds(r0, _RC), :], sem)

    def compute(in_b, out_b):
        def row(r, carry):
            for c in range(_F // 16):
                sl = pl.ds(c * 16, 16)
                out_b[r, sl] = in_b[0, r, sl] + in_b[1, r, sl]
            return carry
        lax.fori_loop(0, _RC, row, 0)

    for k0 in range(_NBI - 1):
        in_copy(k0, k0).start()

    def quad(p, carry):
        for b in range(_NBI):
            k = p * _NBI + b
            u = wid + k * _NW
            valid = u < _UNITS

            @pl.when(valid)
            def _wait_in():
                in_copy(k, b).wait()

            kw = lax.max(k - _NBO, 0)

            @pl.when((k >= _NBO) & (u - _NBO * _NW < _UNITS))
            def _wait_out():
                out_copy(kw, b % _NBO).wait()

            @pl.when(u + (_NBI - 1) * _NW < _UNITS)
            def _prefetch():
                in_copy(k + _NBI - 1, (b + _NBI - 1) % _NBI).start()

            @pl.when(valid)
            def _go():
                compute(ins[b][0], outs[b % _NBO][0])
                out_copy(k, b % _NBO).start()
        return carry

    lax.fori_loop(0, _K // _NBI, quad, 0)


_pair_slab = pl.kernel(
    _pair_slab_body,
    out_type=jax.ShapeDtypeStruct((_G, _B, _F), jnp.float32),
    mesh=plsc.VectorSubcoreMesh(core_axis_name="c", subcore_axis_name="s"),
    compiler_params=pltpu.CompilerParams(needs_layout_passes=False),
    scratch_types=[
        pltpu.VMEM((2, _RC, _F), jnp.float32),
        pltpu.VMEM((2, _RC, _F), jnp.float32),
        pltpu.VMEM((2, _RC, _F), jnp.float32),
        pltpu.VMEM((2, _RC, _F), jnp.float32),
        pltpu.VMEM((_RC, _F), jnp.float32),
        pltpu.VMEM((_RC, _F), jnp.float32),
        pltpu.SemaphoreType.DMA,
        pltpu.SemaphoreType.DMA,
        pltpu.SemaphoreType.DMA,
        pltpu.SemaphoreType.DMA,
        pltpu.SemaphoreType.DMA,
        pltpu.SemaphoreType.DMA,
    ],
)


def kernel(x):
    xt = x.transpose(2, 3, 0, 1).reshape(_G, 2, _B, _F)
    out = _pair_slab(xt)
    return out.reshape(_C, _N // 2, _B, _F).transpose(2, 3, 0, 1)


# speedup vs baseline: 8.5388x; 1.0039x over previous
import jax
import jax.numpy as jnp
from jax import lax
from jax.experimental import pallas as pl
from jax.experimental.pallas import tpu as pltpu
from jax.experimental.pallas import tpu_sc as plsc

_B, _F, _C, _N = 128, 512, 3, 66
_G = (_C * _N) // 2
_RC = 8
_NCHUNK = _B // _RC
_UNITS = _G * _NCHUNK
_NW = 32
_NBI = 4
_NBO = 2
_K = 52


def _pair_slab_body(x_hbm, o_hbm, in0, in1, in2, in3, ot0, ot1,
                    si0, si1, si2, si3, so0, so1):
    wid = lax.axis_index("s") * 2 + lax.axis_index("c")
    ins = ((in0, si0), (in1, si1), (in2, si2), (in3, si3))
    outs = ((ot0, so0), (ot1, so1))

    def unit_coords(k):
        u = wid + k * _NW
        g = lax.shift_right_logical(u, 4)
        r0 = lax.bitwise_and(u, 15) * _RC
        return u, g, r0

    def in_copy(k, slot):
        _, g, r0 = unit_coords(k)
        buf, sem = ins[slot]
        return pltpu.make_async_copy(
            x_hbm.at[g, :, pl.ds(r0, _RC), :], buf, sem)

    def out_copy(k, slot):
        _, g, r0 = unit_coords(k)
        buf, sem = outs[slot]
        return pltpu.make_async_copy(
            buf, o_hbm.at[g, pl.ds(r0, _RC), :], sem)

    def compute(in_b, out_b):
        def row(r, carry):
            for c in range(_F // 16):
                sl = pl.ds(c * 16, 16)
                out_b[r, sl] = in_b[0, r, sl] + in_b[1, r, sl]
            return carry
        lax.fori_loop(0, _RC, row, 0)

    for k0 in range(_NBI - 1):
        in_copy(k0, k0).start()

    def quad(p, carry):
        for b in range(_NBI):
            k = p * _NBI + b
            u = wid + k * _NW
            valid = u < _UNITS

            @pl.when(valid)
            def _wait_in():
                in_copy(k, b).wait()

            kw = lax.max(k - _NBO, 0)

            @pl.when((k >= _NBO) & (u - _NBO * _NW < _UNITS))
            def _wait_out():
                out_copy(kw, b % _NBO).wait()

            @pl.when(u + (_NBI - 1) * _NW < _UNITS)
            def _prefetch():
                in_copy(k + _NBI - 1, (b + _NBI - 1) % _NBI).start()

            @pl.when(valid)
            def _go():
                compute(ins[b][0], outs[b % _NBO][0])
                out_copy(k, b % _NBO).start()
        return carry

    lax.fori_loop(0, _K // _NBI, quad, 0)


_pair_slab = pl.kernel(
    _pair_slab_body,
    out_type=jax.ShapeDtypeStruct((_G, _B, _F), jnp.float32),
    mesh=plsc.VectorSubcoreMesh(core_axis_name="c", subcore_axis_name="s"),
    compiler_params=pltpu.CompilerParams(
        needs_layout_passes=False, skip_device_barrier=True),
    scratch_types=[
        pltpu.VMEM((2, _RC, _F), jnp.float32),
        pltpu.VMEM((2, _RC, _F), jnp.float32),
        pltpu.VMEM((2, _RC, _F), jnp.float32),
        pltpu.VMEM((2, _RC, _F), jnp.float32),
        pltpu.VMEM((_RC, _F), jnp.float32),
        pltpu.VMEM((_RC, _F), jnp.float32),
        pltpu.SemaphoreType.DMA,
        pltpu.SemaphoreType.DMA,
        pltpu.SemaphoreType.DMA,
        pltpu.SemaphoreType.DMA,
        pltpu.SemaphoreType.DMA,
        pltpu.SemaphoreType.DMA,
    ],
)


def kernel(x):
    xt = x.transpose(2, 3, 0, 1).reshape(_G, 2, _B, _F)
    out = _pair_slab(xt)
    return out.reshape(_C, _N // 2, _B, _F).transpose(2, 3, 0, 1)
